# baseline (speedup 1.0000x reference)
import functools

import jax
import jax.numpy as jnp
from jax import lax
from jax.experimental import pallas as pl
from jax.experimental.pallas import tpu as pltpu
from jax.experimental.pallas import tpu_sc as plsc

N_NODE = 100000
DIM = 128
LAMBDA_DIS = 1e-05
B = 500000

NC = 2
NS = 16
LANES = 16
CB = 128
NCH0 = 248
NCH1 = 0
EPW0 = CB * NCH0
EPW1 = CB * NCH1
B_PAD = NS * (EPW0 + EPW1)
PAD = B_PAD - B
R = B_PAD // 128
TOT_CHUNK = B_PAD // CB
KW = DIM // 2 // LANES
NHIST = 102400
HSTRIPE = NHIST // NS
HGRP = 8
N_PAD_ROWS = NHIST - N_NODE
LB = NHIST // 16

_GATHER_DNUMS = lax.GatherDimensionNumbers(
    offset_dims=(), collapsed_slice_dims=(0,), start_index_map=(0,))


def _permute(v, idx):
    return lax.gather(v, idx[:, None], _GATHER_DNUMS, slice_sizes=(1,),
                      mode=lax.GatherScatterMode.PROMISE_IN_BOUNDS)


def _sc_scores_body(table, nid_h, nbr_h, bias_h,
                    scores_o, hist_o,
                    idxA, idxB,
                    rows1a, rows2a, biasa, scoresa,
                    rows1b, rows2b, biasb, scoresb,
                    z_v, ones1,
                    scatN0, scatM0, scatN1, scatM1,
                    hist1_sh, hist2_sh, bias_sh,
                    g1a, g2a, g3a, g1b, g2b, g3b, osema, osemb, hsem):
    c = lax.axis_index("c")
    s = lax.axis_index("s")
    on_core0 = c == 0
    nch = jnp.where(on_core0, NCH0, NCH1)
    base = jnp.where(on_core0, s * EPW0, NS * EPW0 + s * EPW1)
    lane = lax.iota(jnp.int32, LANES)

    @pl.when(on_core0)
    def _():
        pltpu.sync_copy(nid_h.at[pl.ds(base, EPW0)], idxA)
        pltpu.sync_copy(nbr_h.at[pl.ds(base, EPW0)], idxB)

    def zero_step(i, _):
        z_v[pl.ds(i * LANES, LANES)] = jnp.zeros((LANES,), jnp.float32)
        return 0

    lax.fori_loop(0, HSTRIPE // LANES, zero_step, 0)
    pltpu.sync_copy(z_v, hist1_sh.at[pl.ds(s * HSTRIPE, HSTRIPE)])
    pltpu.sync_copy(z_v, hist2_sh.at[pl.ds(s * HSTRIPE, HSTRIPE)])

    @pl.when(s == 0)
    def _():
        pltpu.sync_copy(bias_h, bias_sh.at[pl.ds(0, N_NODE)])
    for q in range(CB // LANES):
        ones1[pl.ds(q * LANES, LANES)] = jnp.full(
            (LANES,), 1.0, jnp.float32)
    plsc.subcore_barrier()

    def start_gathers(cnk, rows1x, rows2x, biasx, s1, s2, s3):
        ia = idxA.at[pl.ds(cnk * CB, CB)]
        ib = idxB.at[pl.ds(cnk * CB, CB)]
        pltpu.make_async_copy(table.at[ia], rows1x, s1).start()
        pltpu.make_async_copy(table.at[ib], rows2x, s2).start()
        pltpu.make_async_copy(bias_sh.at[ib], biasx, s3).start()

    def wait_gathers(cnk, rows1x, rows2x, biasx, s1, s2, s3):
        ia = idxA.at[pl.ds(cnk * CB, CB)]
        ib = idxB.at[pl.ds(cnk * CB, CB)]
        pltpu.make_async_copy(table.at[ia], rows1x, s1).wait()
        pltpu.make_async_copy(table.at[ib], rows2x, s2).wait()
        pltpu.make_async_copy(bias_sh.at[ib], biasx, s3).wait()

    def compute_chunk(rows1x, rows2x, biasx, scoresx):
        def group(g, _):
            bvec = biasx[pl.ds(g * LANES, LANES)]
            block = jnp.zeros((LANES,), jnp.float32)
            for p in range(LANES):
                e = g * LANES + p
                acc0 = None
                acc1 = None
                for k in range(KW):
                    w1 = rows1x[e, pl.ds(k * LANES, LANES)]
                    w2 = rows2x[e, pl.ds(k * LANES, LANES)]
                    v1a = plsc.bitcast(w1 << 16, jnp.float32)
                    v1b = plsc.bitcast(w1, jnp.float32)
                    v2a = plsc.bitcast(w2 << 16, jnp.float32)
                    v2b = plsc.bitcast(w2, jnp.float32)
                    pa = v1a * v2a
                    pb = v1b * v2b
                    acc0 = pa if acc0 is None else acc0 + pa
                    acc1 = pb if acc1 is None else acc1 + pb
                v = acc0 + acc1
                for sh in (8, 4, 2, 1):
                    v = v + _permute(v, lane ^ sh)
                block = jnp.where(lane == p, v, block)
            scoresx[pl.ds(g * LANES, LANES)] = block + bvec
            return 0

        lax.fori_loop(0, CB // LANES, group, 0)

    def pair(i, _):
        ca = 2 * i
        cb = 2 * i + 1
        wait_gathers(ca, rows1a, rows2a, biasa, g1a, g2a, g3a)

        @pl.when(i > 0)
        def _():
            pltpu.make_async_copy(
                scoresa, scores_o.at[pl.ds(base, CB)], osema).wait()

        compute_chunk(rows1a, rows2a, biasa, scoresa)

        @pl.when(ca + 2 < nch)
        def _():
            start_gathers(ca + 2, rows1a, rows2a, biasa, g1a, g2a, g3a)

        pltpu.make_async_copy(
            scoresa, scores_o.at[pl.ds(base + ca * CB, CB)], osema).start()

        wait_gathers(cb, rows1b, rows2b, biasb, g1b, g2b, g3b)

        @pl.when(i > 0)
        def _():
            pltpu.make_async_copy(
                scoresb, scores_o.at[pl.ds(base, CB)], osemb).wait()

        compute_chunk(rows1b, rows2b, biasb, scoresb)

        @pl.when(cb + 2 < nch)
        def _():
            start_gathers(cb + 2, rows1b, rows2b, biasb, g1b, g2b, g3b)

        pltpu.make_async_copy(
            scoresb, scores_o.at[pl.ds(base + cb * CB, CB)], osemb).start()

        return 0

    @pl.when(on_core0)
    def _():
        start_gathers(0, rows1a, rows2a, biasa, g1a, g2a, g3a)
        start_gathers(1, rows1b, rows2b, biasb, g1b, g2b, g3b)
        lax.fori_loop(0, nch // 2, pair, 0)
        pltpu.make_async_copy(
            scoresa, scores_o.at[pl.ds(base, CB)], osema).wait()
        pltpu.make_async_copy(
            scoresb, scores_o.at[pl.ds(base, CB)], osemb).wait()

    def prep(dstN, dstM, cnk):
        for q in range(CB // LANES):
            sl = pl.ds(q * LANES, LANES)
            dstN[sl] = idxA[pl.ds(cnk * CB + q * LANES, LANES)]
            dstM[sl] = idxB[pl.ds(cnk * CB + q * LANES, LANES)]

    def scat_pair(i, _):
        prep(scatN0, scatM0, 2 * i)
        cp1 = pltpu.make_async_copy(ones1, hist1_sh.at[scatN0], hsem)
        cp1.start(add=True)
        cp2 = pltpu.make_async_copy(ones1, hist2_sh.at[scatM0], hsem)
        cp2.start(add=True)
        prep(scatN1, scatM1, 2 * i + 1)
        cp3 = pltpu.make_async_copy(ones1, hist1_sh.at[scatN1], hsem)
        cp3.start(add=True)
        cp4 = pltpu.make_async_copy(ones1, hist2_sh.at[scatM1], hsem)
        cp4.start(add=True)
        for cp in (cp1, cp2, cp3, cp4):
            cp.wait()
        return 0

    @pl.when(on_core0)
    def _():
        lax.fori_loop(0, nch // 2, scat_pair, 0)

    plsc.subcore_barrier()

    @pl.when(s == 0)
    def _():
        pltpu.sync_copy(hist1_sh, hist_o.at[c, 0])
        pltpu.sync_copy(hist2_sh, hist_o.at[c, 1])


_sc_scores = functools.partial(
    pl.kernel,
    mesh=plsc.VectorSubcoreMesh(core_axis_name="c", subcore_axis_name="s"),
    compiler_params=pltpu.CompilerParams(
        needs_layout_passes=False, use_tc_tiling_on_sc=False),
    out_type=[
        jax.ShapeDtypeStruct((B_PAD,), jnp.float32),
        jax.ShapeDtypeStruct((NC, 2, NHIST), jnp.float32),
    ],
    scratch_types=[
        pltpu.VMEM((EPW0,), jnp.int32),
        pltpu.VMEM((EPW0,), jnp.int32),
        pltpu.VMEM((CB, DIM // 2), jnp.int32),
        pltpu.VMEM((CB, DIM // 2), jnp.int32),
        pltpu.VMEM((CB,), jnp.float32),
        pltpu.VMEM((CB,), jnp.float32),
        pltpu.VMEM((CB, DIM // 2), jnp.int32),
        pltpu.VMEM((CB, DIM // 2), jnp.int32),
        pltpu.VMEM((CB,), jnp.float32),
        pltpu.VMEM((CB,), jnp.float32),
        pltpu.VMEM((HSTRIPE,), jnp.float32),
        pltpu.VMEM((CB,), jnp.float32),
        pltpu.VMEM((CB,), jnp.int32),
        pltpu.VMEM((CB,), jnp.int32),
        pltpu.VMEM((CB,), jnp.int32),
        pltpu.VMEM((CB,), jnp.int32),
        pltpu.VMEM_SHARED((NHIST,), jnp.float32),
        pltpu.VMEM_SHARED((NHIST,), jnp.float32),
        pltpu.VMEM_SHARED((NHIST,), jnp.float32),
        pltpu.SemaphoreType.DMA,
        pltpu.SemaphoreType.DMA,
        pltpu.SemaphoreType.DMA,
        pltpu.SemaphoreType.DMA,
        pltpu.SemaphoreType.DMA,
        pltpu.SemaphoreType.DMA,
        pltpu.SemaphoreType.DMA,
        pltpu.SemaphoreType.DMA,
        pltpu.SemaphoreType.DMA,
    ],
)(_sc_scores_body)


def _tc_pack_body(x_ref, out_ref, r_ref):
    x = x_ref[...]
    u = lax.bitcast_convert_type(x, jnp.uint32)
    rb = (u + jnp.uint32(0x7FFF) + ((u >> 16) & jnp.uint32(1))) >> 16
    lo = rb[:, 0:64]
    hi = rb[:, 64:128]
    out_ref[...] = lax.bitcast_convert_type(lo | (hi << 16), jnp.int32)
    gri = (pl.program_id(0) * LB
           + lax.broadcasted_iota(jnp.int32, (LB, DIM), 0))
    xm = jnp.where(gri < N_NODE, x, 0.0)
    ones = jnp.ones((1, DIM), jnp.float32)
    r_ref[...] = lax.dot_general(ones, xm * xm, (((1,), (1,)), ((), ())))


def _tc_l2_body(rrow, hist4, biasrow, out):
    r = rrow[...]
    h = hist4[...]
    cnt_all = h[0:1] + h[1:2] + h[2:3] + h[3:4]
    cnt_nbr = h[1:2] + h[3:4]
    b = biasrow[...]
    part = jnp.sum(cnt_all * r) + jnp.sum(cnt_nbr * b * b)
    out[...] = jnp.reshape(part, (1, 1))


def _tc_combine_body(scores, label, l2in, emb0, bias0, out):
    sarr = scores[...]
    y = label[...].astype(jnp.float32)
    pos = (lax.broadcasted_iota(jnp.int32, (R, 128), 0) * 128
           + lax.broadcasted_iota(jnp.int32, (R, 128), 1))
    validf = (pos < B).astype(jnp.float32)
    prob = jax.nn.sigmoid(sarr)
    eps = 1e-12
    ll = (y * jnp.log(jnp.clip(prob, eps, 1.0))
          + (1.0 - y) * jnp.log(jnp.clip(1.0 - prob, eps, 1.0)))
    bce = -jnp.sum(ll * validf) / B
    e0 = emb0[...]
    col0 = (lax.broadcasted_iota(jnp.int32, (1, 128), 1) == 0)
    b0 = bias0[...] * col0.astype(jnp.float32)
    corr = float(PAD) * (2.0 * jnp.sum(e0 * e0) + jnp.sum(b0 * b0))
    l2 = l2in[0, 0] - corr
    total = bce + l2 * (0.5 * LAMBDA_DIS)
    out[...] = jnp.reshape(total, (1, 1))


def _pack_table(embedding_matrix):
    return pl.pallas_call(
        _tc_pack_body,
        grid=(16,),
        in_specs=[pl.BlockSpec((LB, DIM), lambda i: (i, 0))],
        out_specs=[
            pl.BlockSpec((LB, DIM // 2), lambda i: (i, 0)),
            pl.BlockSpec((1, LB), lambda i: (0, i)),
        ],
        out_shape=[
            jax.ShapeDtypeStruct((N_NODE, DIM // 2), jnp.int32),
            jax.ShapeDtypeStruct((1, NHIST), jnp.float32),
        ],
    )(embedding_matrix)


def kernel(node_id, node_neighbor_id, label, embedding_matrix, bias):
    zi = jnp.zeros((PAD,), jnp.int32)
    nid = jnp.concatenate([node_id, zi])
    nbr = jnp.concatenate([node_neighbor_id, zi])
    lab = jnp.concatenate([label, zi])
    table_i, rrow = _pack_table(embedding_matrix)
    scores, hist = _sc_scores(table_i, nid, nbr, bias)

    biasrow = jnp.concatenate(
        [bias, jnp.zeros((NHIST - N_NODE,), jnp.float32)]).reshape(1, NHIST)
    hist4 = hist.reshape(4, NHIST)
    l2 = pl.pallas_call(
        _tc_l2_body,
        out_shape=jax.ShapeDtypeStruct((1, 1), jnp.float32),
    )(rrow, hist4, biasrow)

    emb0 = embedding_matrix[0:1, :]
    bias0 = bias[0:128].reshape(1, 128)
    out = pl.pallas_call(
        _tc_combine_body,
        out_shape=jax.ShapeDtypeStruct((1, 1), jnp.float32),
    )(scores.reshape(R, 128), lab.reshape(R, 128), l2, emb0, bias0)
    return out[0, 0]


# speedup vs baseline: 1.4879x; 1.4879x over previous
import functools

import jax
import jax.numpy as jnp
from jax import lax
from jax.experimental import pallas as pl
from jax.experimental.pallas import tpu as pltpu
from jax.experimental.pallas import tpu_sc as plsc

N_NODE = 100000
DIM = 128
LAMBDA_DIS = 1e-05
B = 500000

NC = 2
NS = 16
LANES = 16
CB = 128
NCH0 = 232
NCH1 = 16
EPW0 = CB * NCH0
EPW1 = CB * NCH1
B_PAD = NS * (EPW0 + EPW1)
PAD = B_PAD - B
R = B_PAD // 128
TOT_CHUNK = B_PAD // CB
KW = DIM // 2 // LANES
NHIST = 102400
HSTRIPE = NHIST // NS
HGRP = 8
N_PAD_ROWS = NHIST - N_NODE
LB = NHIST // 16

_GATHER_DNUMS = lax.GatherDimensionNumbers(
    offset_dims=(), collapsed_slice_dims=(0,), start_index_map=(0,))


def _permute(v, idx):
    return lax.gather(v, idx[:, None], _GATHER_DNUMS, slice_sizes=(1,),
                      mode=lax.GatherScatterMode.PROMISE_IN_BOUNDS)


def _sc_scores_body(table, nid_h, nbr_h, bias_h,
                    scores_o, hist_o,
                    idxA, idxB,
                    rows1a, rows2a, biasa, scoresa,
                    rows1b, rows2b, biasb, scoresb,
                    z_v, ones1,
                    scatN0, scatM0, scatN1, scatM1,
                    hist1_sh, hist2_sh, bias_sh,
                    g1a, g2a, g3a, g1b, g2b, g3b, osema, osemb, hsem):
    c = lax.axis_index("c")
    s = lax.axis_index("s")
    on_core0 = c == 0
    nch = jnp.where(on_core0, NCH0, NCH1)
    base = jnp.where(on_core0, s * EPW0, NS * EPW0 + s * EPW1)
    lane = lax.iota(jnp.int32, LANES)

    @pl.when(on_core0)
    def _():
        pltpu.sync_copy(nid_h.at[pl.ds(base, EPW0)], idxA)
        pltpu.sync_copy(nbr_h.at[pl.ds(base, EPW0)], idxB)

    @pl.when(jnp.logical_not(on_core0))
    def _():
        pltpu.sync_copy(nid_h.at[pl.ds(base, EPW1)], idxA.at[pl.ds(0, EPW1)])
        pltpu.sync_copy(nbr_h.at[pl.ds(base, EPW1)], idxB.at[pl.ds(0, EPW1)])

    def zero_step(i, _):
        z_v[pl.ds(i * LANES, LANES)] = jnp.zeros((LANES,), jnp.float32)
        return 0

    lax.fori_loop(0, HSTRIPE // LANES, zero_step, 0)
    pltpu.sync_copy(z_v, hist1_sh.at[pl.ds(s * HSTRIPE, HSTRIPE)])
    pltpu.sync_copy(z_v, hist2_sh.at[pl.ds(s * HSTRIPE, HSTRIPE)])

    @pl.when(s == 0)
    def _():
        pltpu.sync_copy(bias_h, bias_sh.at[pl.ds(0, N_NODE)])
    for q in range(CB // LANES):
        ones1[pl.ds(q * LANES, LANES)] = jnp.full(
            (LANES,), 1.0, jnp.float32)
    plsc.subcore_barrier()

    def start_gathers(cnk, rows1x, rows2x, biasx, s1, s2, s3):
        ia = idxA.at[pl.ds(cnk * CB, CB)]
        ib = idxB.at[pl.ds(cnk * CB, CB)]
        pltpu.make_async_copy(table.at[ia], rows1x, s1).start()
        pltpu.make_async_copy(table.at[ib], rows2x, s2).start()
        pltpu.make_async_copy(bias_sh.at[ib], biasx, s3).start()

    def wait_gathers(cnk, rows1x, rows2x, biasx, s1, s2, s3):
        ia = idxA.at[pl.ds(cnk * CB, CB)]
        ib = idxB.at[pl.ds(cnk * CB, CB)]
        pltpu.make_async_copy(table.at[ia], rows1x, s1).wait()
        pltpu.make_async_copy(table.at[ib], rows2x, s2).wait()
        pltpu.make_async_copy(bias_sh.at[ib], biasx, s3).wait()

    def compute_chunk(rows1x, rows2x, biasx, scoresx):
        def group(g, _):
            bvec = biasx[pl.ds(g * LANES, LANES)]
            block = jnp.zeros((LANES,), jnp.float32)
            for p in range(LANES):
                e = g * LANES + p
                acc0 = None
                acc1 = None
                for k in range(KW):
                    w1 = rows1x[e, pl.ds(k * LANES, LANES)]
                    w2 = rows2x[e, pl.ds(k * LANES, LANES)]
                    v1a = plsc.bitcast(w1 << 16, jnp.float32)
                    v1b = plsc.bitcast(w1, jnp.float32)
                    v2a = plsc.bitcast(w2 << 16, jnp.float32)
                    v2b = plsc.bitcast(w2, jnp.float32)
                    pa = v1a * v2a
                    pb = v1b * v2b
                    acc0 = pa if acc0 is None else acc0 + pa
                    acc1 = pb if acc1 is None else acc1 + pb
                v = acc0 + acc1
                for sh in (8, 4, 2, 1):
                    v = v + _permute(v, lane ^ sh)
                block = jnp.where(lane == p, v, block)
            scoresx[pl.ds(g * LANES, LANES)] = block + bvec
            return 0

        lax.fori_loop(0, CB // LANES, group, 0)

    def pair(i, _):
        ca = 2 * i
        cb = 2 * i + 1
        wait_gathers(ca, rows1a, rows2a, biasa, g1a, g2a, g3a)

        @pl.when(i > 0)
        def _():
            pltpu.make_async_copy(
                scoresa, scores_o.at[pl.ds(base, CB)], osema).wait()

        compute_chunk(rows1a, rows2a, biasa, scoresa)

        @pl.when(ca + 2 < nch)
        def _():
            start_gathers(ca + 2, rows1a, rows2a, biasa, g1a, g2a, g3a)

        pltpu.make_async_copy(
            scoresa, scores_o.at[pl.ds(base + ca * CB, CB)], osema).start()

        wait_gathers(cb, rows1b, rows2b, biasb, g1b, g2b, g3b)

        @pl.when(i > 0)
        def _():
            pltpu.make_async_copy(
                scoresb, scores_o.at[pl.ds(base, CB)], osemb).wait()

        compute_chunk(rows1b, rows2b, biasb, scoresb)

        @pl.when(cb + 2 < nch)
        def _():
            start_gathers(cb + 2, rows1b, rows2b, biasb, g1b, g2b, g3b)

        pltpu.make_async_copy(
            scoresb, scores_o.at[pl.ds(base + cb * CB, CB)], osemb).start()

        return 0

    start_gathers(0, rows1a, rows2a, biasa, g1a, g2a, g3a)
    start_gathers(1, rows1b, rows2b, biasb, g1b, g2b, g3b)
    lax.fori_loop(0, nch // 2, pair, 0)
    pltpu.make_async_copy(scoresa, scores_o.at[pl.ds(base, CB)], osema).wait()
    pltpu.make_async_copy(scoresb, scores_o.at[pl.ds(base, CB)], osemb).wait()

    def prep(dstN, dstM, cnk):
        for q in range(CB // LANES):
            sl = pl.ds(q * LANES, LANES)
            dstN[sl] = idxA[pl.ds(cnk * CB + q * LANES, LANES)]
            dstM[sl] = idxB[pl.ds(cnk * CB + q * LANES, LANES)]

    def scat_pair(i, _):
        prep(scatN0, scatM0, 2 * i)
        cp1 = pltpu.make_async_copy(ones1, hist1_sh.at[scatN0], hsem)
        cp1.start(add=True)
        cp2 = pltpu.make_async_copy(ones1, hist2_sh.at[scatM0], hsem)
        cp2.start(add=True)
        prep(scatN1, scatM1, 2 * i + 1)
        cp3 = pltpu.make_async_copy(ones1, hist1_sh.at[scatN1], hsem)
        cp3.start(add=True)
        cp4 = pltpu.make_async_copy(ones1, hist2_sh.at[scatM1], hsem)
        cp4.start(add=True)
        for cp in (cp1, cp2, cp3, cp4):
            cp.wait()
        return 0

    lax.fori_loop(0, nch // 2, scat_pair, 0)
    plsc.subcore_barrier()

    @pl.when(s == 0)
    def _():
        pltpu.sync_copy(hist1_sh, hist_o.at[c, 0])
        pltpu.sync_copy(hist2_sh, hist_o.at[c, 1])


_sc_scores = functools.partial(
    pl.kernel,
    mesh=plsc.VectorSubcoreMesh(core_axis_name="c", subcore_axis_name="s"),
    compiler_params=pltpu.CompilerParams(
        needs_layout_passes=False, use_tc_tiling_on_sc=False),
    out_type=[
        jax.ShapeDtypeStruct((B_PAD,), jnp.float32),
        jax.ShapeDtypeStruct((NC, 2, NHIST), jnp.float32),
    ],
    scratch_types=[
        pltpu.VMEM((EPW0,), jnp.int32),
        pltpu.VMEM((EPW0,), jnp.int32),
        pltpu.VMEM((CB, DIM // 2), jnp.int32),
        pltpu.VMEM((CB, DIM // 2), jnp.int32),
        pltpu.VMEM((CB,), jnp.float32),
        pltpu.VMEM((CB,), jnp.float32),
        pltpu.VMEM((CB, DIM // 2), jnp.int32),
        pltpu.VMEM((CB, DIM // 2), jnp.int32),
        pltpu.VMEM((CB,), jnp.float32),
        pltpu.VMEM((CB,), jnp.float32),
        pltpu.VMEM((HSTRIPE,), jnp.float32),
        pltpu.VMEM((CB,), jnp.float32),
        pltpu.VMEM((CB,), jnp.int32),
        pltpu.VMEM((CB,), jnp.int32),
        pltpu.VMEM((CB,), jnp.int32),
        pltpu.VMEM((CB,), jnp.int32),
        pltpu.VMEM_SHARED((NHIST,), jnp.float32),
        pltpu.VMEM_SHARED((NHIST,), jnp.float32),
        pltpu.VMEM_SHARED((NHIST,), jnp.float32),
        pltpu.SemaphoreType.DMA,
        pltpu.SemaphoreType.DMA,
        pltpu.SemaphoreType.DMA,
        pltpu.SemaphoreType.DMA,
        pltpu.SemaphoreType.DMA,
        pltpu.SemaphoreType.DMA,
        pltpu.SemaphoreType.DMA,
        pltpu.SemaphoreType.DMA,
        pltpu.SemaphoreType.DMA,
    ],
)(_sc_scores_body)


def _tc_pack_body(x_ref, out_ref, r_ref):
    x = x_ref[...]
    u = lax.bitcast_convert_type(x, jnp.uint32)
    rb = (u + jnp.uint32(0x7FFF) + ((u >> 16) & jnp.uint32(1))) >> 16
    lo = rb[:, 0:64]
    hi = rb[:, 64:128]
    out_ref[...] = lax.bitcast_convert_type(lo | (hi << 16), jnp.int32)
    gri = (pl.program_id(0) * LB
           + lax.broadcasted_iota(jnp.int32, (LB, DIM), 0))
    xm = jnp.where(gri < N_NODE, x, 0.0)
    ones = jnp.ones((1, DIM), jnp.float32)
    r_ref[...] = lax.dot_general(ones, xm * xm, (((1,), (1,)), ((), ())))


def _tc_l2_body(rrow, hist4, biasrow, out):
    r = rrow[...]
    h = hist4[...]
    cnt_all = h[0:1] + h[1:2] + h[2:3] + h[3:4]
    cnt_nbr = h[1:2] + h[3:4]
    b = biasrow[...]
    part = jnp.sum(cnt_all * r) + jnp.sum(cnt_nbr * b * b)
    out[...] = jnp.reshape(part, (1, 1))


def _tc_combine_body(scores, label, l2in, emb0, bias0, out):
    sarr = scores[...]
    y = label[...].astype(jnp.float32)
    pos = (lax.broadcasted_iota(jnp.int32, (R, 128), 0) * 128
           + lax.broadcasted_iota(jnp.int32, (R, 128), 1))
    validf = (pos < B).astype(jnp.float32)
    prob = jax.nn.sigmoid(sarr)
    eps = 1e-12
    ll = (y * jnp.log(jnp.clip(prob, eps, 1.0))
          + (1.0 - y) * jnp.log(jnp.clip(1.0 - prob, eps, 1.0)))
    bce = -jnp.sum(ll * validf) / B
    e0 = emb0[...]
    col0 = (lax.broadcasted_iota(jnp.int32, (1, 128), 1) == 0)
    b0 = bias0[...] * col0.astype(jnp.float32)
    corr = float(PAD) * (2.0 * jnp.sum(e0 * e0) + jnp.sum(b0 * b0))
    l2 = l2in[0, 0] - corr
    total = bce + l2 * (0.5 * LAMBDA_DIS)
    out[...] = jnp.reshape(total, (1, 1))


def _pack_table(embedding_matrix):
    return pl.pallas_call(
        _tc_pack_body,
        grid=(16,),
        in_specs=[pl.BlockSpec((LB, DIM), lambda i: (i, 0))],
        out_specs=[
            pl.BlockSpec((LB, DIM // 2), lambda i: (i, 0)),
            pl.BlockSpec((1, LB), lambda i: (0, i)),
        ],
        out_shape=[
            jax.ShapeDtypeStruct((N_NODE, DIM // 2), jnp.int32),
            jax.ShapeDtypeStruct((1, NHIST), jnp.float32),
        ],
    )(embedding_matrix)


def kernel(node_id, node_neighbor_id, label, embedding_matrix, bias):
    zi = jnp.zeros((PAD,), jnp.int32)
    nid = jnp.concatenate([node_id, zi])
    nbr = jnp.concatenate([node_neighbor_id, zi])
    lab = jnp.concatenate([label, zi])
    table_i, rrow = _pack_table(embedding_matrix)
    scores, hist = _sc_scores(table_i, nid, nbr, bias)

    biasrow = jnp.concatenate(
        [bias, jnp.zeros((NHIST - N_NODE,), jnp.float32)]).reshape(1, NHIST)
    hist4 = hist.reshape(4, NHIST)
    l2 = pl.pallas_call(
        _tc_l2_body,
        out_shape=jax.ShapeDtypeStruct((1, 1), jnp.float32),
    )(rrow, hist4, biasrow)

    emb0 = embedding_matrix[0:1, :]
    bias0 = bias[0:128].reshape(1, 128)
    out = pl.pallas_call(
        _tc_combine_body,
        out_shape=jax.ShapeDtypeStruct((1, 1), jnp.float32),
    )(scores.reshape(R, 128), lab.reshape(R, 128), l2, emb0, bias0)
    return out[0, 0]
